# baseline (speedup 1.0000x reference)
import jax
import jax.numpy as jnp
from jax.experimental import pallas as pl
from jax.experimental.pallas import tpu as pltpu

N_EMBD = 768
N_HEAD = 12
HEAD_SIZE = 64
N_EXPERTS = 3
F = 4 * N_EMBD
T = 2048

QT = 512
MT = 256
FC = 1536
NF = F // FC
NEG = -1e30
NT_Q = T // QT
NT_M = T // MT


def _dot(a, b, dims=None):
    if dims is None:
        return jax.lax.dot(a, b, preferred_element_type=jnp.float32)
    return jax.lax.dot_general(a, b, dims,
                               preferred_element_type=jnp.float32)


def _rms(x, w, eps=1e-6):
    return x * jax.lax.rsqrt(jnp.mean(x * x, axis=-1, keepdims=True) + eps) * w


def _qkv_kernel(x_ref, w_ref, wqkv_ref, o_ref):
    xn = _rms(x_ref[...], w_ref[...])
    o_ref[...] = _dot(xn, wqkv_ref[...])


def _attn_gate_kernel(q_ref, k_ref, v_ref, x_ref, wo_ref, bo_ref,
                      ln2_ref, wg_ref, x1_ref, mask_ref, aux_ref,
                      stat_ref):
    t = pl.program_id(0)
    qq = q_ref[...]
    kk = k_ref[...]
    vv = v_ref[...]
    outs = []
    for h in range(N_HEAD):
        sl = slice(h * HEAD_SIZE, (h + 1) * HEAD_SIZE)
        q = qq[:, sl]
        k = kk[:, sl]
        v = vv[:, sl]
        s = _dot(q, k, (((1,), (1,)), ((), ()))) * 0.125
        m = jnp.max(s, axis=-1, keepdims=True)
        p = jnp.exp(s - m)
        l = jnp.sum(p, axis=-1, keepdims=True)
        outs.append(_dot(p, v) / l)
    att = jnp.concatenate(outs, axis=1)
    x1 = x_ref[...] + bo_ref[...] + _dot(att, wo_ref[...])
    x1_ref[...] = x1
    h2 = _rms(x1, ln2_ref[...])
    col = jax.lax.broadcasted_iota(jnp.int32, (1, 128), 1)
    logits = _dot(h2, wg_ref[...]) + jnp.where(col < N_EXPERTS, 0.0, NEG)
    m = jnp.max(logits, axis=-1, keepdims=True)
    e = jnp.exp(logits - m)
    probs = e / jnp.sum(e, axis=-1, keepdims=True)
    p0 = probs[:, 0:1]
    p1 = probs[:, 1:2]
    p2 = probs[:, 2:3]
    ex0 = (p1 > p0) & (p2 > p0)
    ex1 = (p0 >= p1) & (p2 > p1)
    ex2 = (p0 >= p2) & (p1 >= p2)
    pex = jnp.where(ex0, p0, jnp.where(ex1, p1, p2))
    denom = (p0 + p1 + p2) - pex
    m0 = jnp.where(ex0, 0.0, p0 / denom)
    m1 = jnp.where(ex1, 0.0, p1 / denom)
    m2 = jnp.where(ex2, 0.0, p2 / denom)
    mask_ref[...] = (jnp.where(col == 0, m0, 0.0) +
                     jnp.where(col == 1, m1, 0.0) +
                     jnp.where(col == 2, m2, 0.0))
    t0 = (p0 >= p1) & (p0 >= p2)
    t1 = jnp.logical_not(t0) & (p1 >= p2)
    t2 = jnp.logical_not(t0) & jnp.logical_not(t1)
    part = (jnp.where(col == 0, jnp.sum(p0), 0.0) +
            jnp.where(col == 1, jnp.sum(p1), 0.0) +
            jnp.where(col == 2, jnp.sum(p2), 0.0) +
            jnp.where(col == 3, jnp.sum(t0.astype(jnp.float32)), 0.0) +
            jnp.where(col == 4, jnp.sum(t1.astype(jnp.float32)), 0.0) +
            jnp.where(col == 5, jnp.sum(t2.astype(jnp.float32)), 0.0))
    prev = jnp.where(t == 0, jnp.zeros_like(part), stat_ref[...])
    stat = prev + part
    stat_ref[...] = stat

    @pl.when(t == NT_Q - 1)
    def _():
        imp = stat[:, 0:3]
        load = stat[:, 3:6]
        aux = (N_EXPERTS * 0.01 / (T * T)) * jnp.sum(imp * load)
        aux_ref[...] = jnp.full((1, 1), 1.0, jnp.float32) * aux


def _moe_kernel(x_ref, w_ref, mask_ref, w1_ref, b1_ref, w2_ref, b2_ref,
                o_ref, acc_ref, h2s_ref):
    e = pl.program_id(0)
    fc = pl.program_id(1)
    t = pl.program_id(2)
    first = jnp.logical_and(e == 0, fc == 0)
    last = jnp.logical_and(e == N_EXPERTS - 1, fc == NF - 1)
    rows = pl.ds(t * MT, MT)
    x = x_ref[...]
    h2_new = _rms(x, w_ref[...])
    h2 = jnp.where(first, h2_new, h2s_ref[rows, :])
    hid = jnp.maximum(
        jax.lax.dot(h2.astype(jnp.bfloat16), w1_ref[0],
                    preferred_element_type=jnp.float32) + b1_ref[0], 0.0)
    part = jax.lax.dot(hid.astype(jnp.bfloat16), w2_ref[0],
                       preferred_element_type=jnp.float32)
    part = part + jnp.where(fc == 0, 1.0, 0.0) * b2_ref[0]
    col = jax.lax.broadcasted_iota(jnp.int32, (1, 128), 1)
    msel = jnp.sum(mask_ref[...] * (col == e).astype(jnp.float32),
                   axis=-1, keepdims=True)
    contrib = msel * part
    prev = jnp.where(first, x, acc_ref[rows, :])
    new = prev + contrib
    acc_ref[rows, :] = new

    @pl.when(first)
    def _():
        h2s_ref[rows, :] = h2_new

    @pl.when(last)
    def _():
        o_ref[...] = new


def kernel(x, ln1_w, ln2_w, Wq, Wk, Wv, Wo, bo, Wg, W1, b1, W2, b2):
    x2 = x.reshape(T, N_EMBD)
    ln1 = ln1_w.reshape(1, N_EMBD)
    ln2 = ln2_w.reshape(1, N_EMBD)
    bo2 = bo.reshape(1, N_EMBD)
    wqkv = jnp.concatenate([
        jnp.transpose(Wq, (1, 0, 2)).reshape(N_EMBD, N_HEAD * HEAD_SIZE),
        jnp.transpose(Wk, (1, 0, 2)).reshape(N_EMBD, N_HEAD * HEAD_SIZE),
        jnp.transpose(Wv, (1, 0, 2)).reshape(N_EMBD, N_HEAD * HEAD_SIZE),
    ], axis=1)
    wg_pad = jnp.pad(Wg, ((0, 0), (0, 128 - N_EXPERTS)))

    qkv = pl.pallas_call(
        _qkv_kernel,
        grid=(T // QT,),
        in_specs=[
            pl.BlockSpec((QT, N_EMBD), lambda t: (t, 0)),
            pl.BlockSpec((1, N_EMBD), lambda t: (0, 0)),
            pl.BlockSpec((N_EMBD, 3 * N_EMBD), lambda t: (0, 0)),
        ],
        out_specs=pl.BlockSpec((QT, 3 * N_EMBD), lambda t: (t, 0)),
        out_shape=jax.ShapeDtypeStruct((T, 3 * N_EMBD), jnp.float32),
    )(x2, ln1, wqkv)

    x1, mask, aux = pl.pallas_call(
        _attn_gate_kernel,
        grid=(NT_Q,),
        in_specs=[
            pl.BlockSpec((QT, N_EMBD), lambda t: (t, 0)),
            pl.BlockSpec((T, N_EMBD), lambda t: (0, 1)),
            pl.BlockSpec((T, N_EMBD), lambda t: (0, 2)),
            pl.BlockSpec((QT, N_EMBD), lambda t: (t, 0)),
            pl.BlockSpec((N_EMBD, N_EMBD), lambda t: (0, 0)),
            pl.BlockSpec((1, N_EMBD), lambda t: (0, 0)),
            pl.BlockSpec((1, N_EMBD), lambda t: (0, 0)),
            pl.BlockSpec((N_EMBD, 128), lambda t: (0, 0)),
        ],
        out_specs=[
            pl.BlockSpec((QT, N_EMBD), lambda t: (t, 0)),
            pl.BlockSpec((QT, 128), lambda t: (t, 0)),
            pl.BlockSpec((1, 1), lambda t: (0, 0)),
        ],
        out_shape=[
            jax.ShapeDtypeStruct((T, N_EMBD), jnp.float32),
            jax.ShapeDtypeStruct((T, 128), jnp.float32),
            jax.ShapeDtypeStruct((1, 1), jnp.float32),
        ],
        scratch_shapes=[pltpu.VMEM((1, 128), jnp.float32)],
    )(qkv, qkv, qkv, x2, Wo, bo2, ln2, wg_pad)

    out = pl.pallas_call(
        _moe_kernel,
        grid=(N_EXPERTS, NF, NT_M),
        in_specs=[
            pl.BlockSpec((MT, N_EMBD),
                         lambda e, f, t:
                         (jnp.where((e == 0) & (f == 0), t, 0), 0)),
            pl.BlockSpec((1, N_EMBD), lambda e, f, t: (0, 0)),
            pl.BlockSpec((MT, 128), lambda e, f, t: (t, 0)),
            pl.BlockSpec((1, N_EMBD, FC), lambda e, f, t: (e, 0, f)),
            pl.BlockSpec((1, 1, FC), lambda e, f, t: (e, 0, f)),
            pl.BlockSpec((1, FC, N_EMBD), lambda e, f, t: (e, f, 0)),
            pl.BlockSpec((1, 1, N_EMBD), lambda e, f, t: (e, 0, 0)),
        ],
        out_specs=pl.BlockSpec(
            (MT, N_EMBD),
            lambda e, f, t:
            (jnp.where((e == N_EXPERTS - 1) & (f == NF - 1), t, 0), 0)),
        out_shape=jax.ShapeDtypeStruct((T, N_EMBD), jnp.float32),
        scratch_shapes=[pltpu.VMEM((T, N_EMBD), jnp.float32),
                        pltpu.VMEM((T, N_EMBD), jnp.float32)],
    )(x1, ln2, mask, W1.astype(jnp.bfloat16), b1.reshape(N_EXPERTS, 1, F),
      W2.astype(jnp.bfloat16), b2.reshape(N_EXPERTS, 1, N_EMBD))

    return (out.reshape(1, T, N_EMBD), aux.reshape(()))


# speedup vs baseline: 1.0381x; 1.0381x over previous
import jax
import jax.numpy as jnp
from jax.experimental import pallas as pl
from jax.experimental.pallas import tpu as pltpu
from jax.experimental.pallas import tpu_sc as plsc
import functools

N_EMBD = 768
N_HEAD = 12
HEAD_SIZE = 64
N_EXPERTS = 3
F = 4 * N_EMBD
T = 2048

QT = 512
MT = 256
FC = 1536
NF = F // FC
NEG = -1e30
NT_Q = T // QT
NT_M = T // MT


def _dot(a, b, dims=None):
    if dims is None:
        return jax.lax.dot(a, b, preferred_element_type=jnp.float32)
    return jax.lax.dot_general(a, b, dims,
                               preferred_element_type=jnp.float32)


def _rms(x, w, eps=1e-6):
    return x * jax.lax.rsqrt(jnp.mean(x * x, axis=-1, keepdims=True) + eps) * w


def _qkv_kernel(x_ref, w_ref, wqkv_ref, o_ref):
    xn = _rms(x_ref[...], w_ref[...])
    o_ref[...] = _dot(xn, wqkv_ref[...])


def _attn_gate_kernel(q_ref, k_ref, v_ref, x_ref, wo_ref, bo_ref,
                      ln2_ref, wg_ref, x1_ref, mask_ref, probs_ref):
    qq = q_ref[...]
    kk = k_ref[...]
    vv = v_ref[...]
    outs = []
    for h in range(N_HEAD):
        sl = slice(h * HEAD_SIZE, (h + 1) * HEAD_SIZE)
        q = qq[:, sl]
        k = kk[:, sl]
        v = vv[:, sl]
        s = _dot(q, k, (((1,), (1,)), ((), ()))) * 0.125
        m = jnp.max(s, axis=-1, keepdims=True)
        p = jnp.exp(s - m)
        l = jnp.sum(p, axis=-1, keepdims=True)
        outs.append(_dot(p, v) / l)
    att = jnp.concatenate(outs, axis=1)
    x1 = x_ref[...] + bo_ref[...] + _dot(att, wo_ref[...])
    x1_ref[...] = x1
    h2 = _rms(x1, ln2_ref[...])
    col = jax.lax.broadcasted_iota(jnp.int32, (1, 128), 1)
    logits = _dot(h2, wg_ref[...]) + jnp.where(col < N_EXPERTS, 0.0, NEG)
    m = jnp.max(logits, axis=-1, keepdims=True)
    e = jnp.exp(logits - m)
    probs = e / jnp.sum(e, axis=-1, keepdims=True)
    p0 = probs[:, 0:1]
    p1 = probs[:, 1:2]
    p2 = probs[:, 2:3]
    ex0 = (p1 > p0) & (p2 > p0)
    ex1 = (p0 >= p1) & (p2 > p1)
    ex2 = (p0 >= p2) & (p1 >= p2)
    pex = jnp.where(ex0, p0, jnp.where(ex1, p1, p2))
    denom = (p0 + p1 + p2) - pex
    m0 = jnp.where(ex0, 0.0, p0 / denom)
    m1 = jnp.where(ex1, 0.0, p1 / denom)
    m2 = jnp.where(ex2, 0.0, p2 / denom)
    mask_ref[...] = (jnp.where(col == 0, m0, 0.0) +
                     jnp.where(col == 1, m1, 0.0) +
                     jnp.where(col == 2, m2, 0.0))
    probs_ref[...] = probs[:, 0:8]


def _sc_aux(probs_tr_flat):
    info = plsc.get_sparse_core_info()
    nsub = info.num_subcores
    per = T // nsub
    nch = per // 16
    coef = N_EXPERTS * 0.01 / (T * T)
    mesh = plsc.VectorSubcoreMesh(core_axis_name="c", subcore_axis_name="s")

    @functools.partial(
        pl.kernel, mesh=mesh,
        out_type=jax.ShapeDtypeStruct((16,), jnp.float32),
        scratch_types=[
            pltpu.VMEM((per,), jnp.float32),
            pltpu.VMEM((per,), jnp.float32),
            pltpu.VMEM((per,), jnp.float32),
            pltpu.VMEM((96,), jnp.float32),
            pltpu.VMEM((nsub * 96,), jnp.float32),
            pltpu.VMEM((16,), jnp.float32),
            pltpu.VMEM_SHARED((nsub * 96,), jnp.float32),
        ],
    )
    def body(probs_hbm, out_hbm, p0v, p1v, p2v, six_v, gath_v, out_v,
             shared):
        cid = jax.lax.axis_index("c")
        sid = jax.lax.axis_index("s")
        base = sid * per
        pltpu.sync_copy(probs_hbm.at[pl.ds(base, per)], p0v)
        pltpu.sync_copy(probs_hbm.at[pl.ds(T + base, per)], p1v)
        pltpu.sync_copy(probs_hbm.at[pl.ds(2 * T + base, per)], p2v)
        iota = jax.lax.iota(jnp.int32, 16)
        zero = jnp.zeros((16,), jnp.float32)
        one = jnp.ones((16,), jnp.float32)
        sp0 = sp1 = sp2 = st0 = st1 = st2 = zero
        for i in range(nch):
            sl = pl.ds(16 * i, 16)
            p0 = p0v[sl]
            p1 = p1v[sl]
            p2 = p2v[sl]
            sp0 = sp0 + p0
            sp1 = sp1 + p1
            sp2 = sp2 + p2
            g01 = jnp.where(p0 >= p1, one, zero)
            g02 = jnp.where(p0 >= p2, one, zero)
            g12 = jnp.where(p1 >= p2, one, zero)
            t0 = g01 * g02
            t1 = (one - t0) * g12
            t2 = one - t0 - t1
            st0 = st0 + t0
            st1 = st1 + t1
            st2 = st2 + t2
        for k, v in enumerate((sp0, sp1, sp2, st0, st1, st2)):
            six_v[pl.ds(k * 16, 16)] = v
        pltpu.sync_copy(six_v, shared.at[pl.ds(sid * 96, 96)])
        plsc.subcore_barrier()

        @pl.when(jnp.logical_and(cid == 0, sid == 0))
        def _():
            pltpu.sync_copy(shared, gath_v)
            for k in range(6):
                acc = gath_v[pl.ds(k * 16, 16)]
                for j in range(1, nsub):
                    acc = acc + gath_v[pl.ds(j * 96 + k * 16, 16)]
                six_v[pl.ds(k * 16, 16)] = acc
            sums = []
            for k in range(6):
                v = six_v[pl.ds(k * 16, 16)]
                s = v[0]
                for l in range(1, 16):
                    s = s + v[l]
                sums.append(s)
            aux = (sums[0] * sums[3] + sums[1] * sums[4] +
                   sums[2] * sums[5]) * coef
            out_v[...] = jnp.where(iota == 0, aux, 0.0)
            pltpu.sync_copy(out_v, out_hbm)

    return body(probs_tr_flat)


def _moe_kernel(x_ref, w_ref, mask_ref, w1_ref, b1_ref, w2_ref, b2_ref,
                o_ref, acc_ref, h2s_ref):
    e = pl.program_id(0)
    fc = pl.program_id(1)
    t = pl.program_id(2)
    first = jnp.logical_and(e == 0, fc == 0)
    last = jnp.logical_and(e == N_EXPERTS - 1, fc == NF - 1)
    rows = pl.ds(t * MT, MT)
    x = x_ref[...]
    h2_new = _rms(x, w_ref[...])
    h2 = jnp.where(first, h2_new, h2s_ref[rows, :])
    hid = jnp.maximum(_dot(h2, w1_ref[0]) + b1_ref[0], 0.0)
    part = _dot(hid, w2_ref[0])
    part = part + jnp.where(fc == 0, 1.0, 0.0) * b2_ref[0]
    col = jax.lax.broadcasted_iota(jnp.int32, (1, 128), 1)
    msel = jnp.sum(mask_ref[...] * (col == e).astype(jnp.float32),
                   axis=-1, keepdims=True)
    contrib = msel * part
    prev = jnp.where(first, x, acc_ref[rows, :])
    new = prev + contrib
    acc_ref[rows, :] = new

    @pl.when(first)
    def _():
        h2s_ref[rows, :] = h2_new

    @pl.when(last)
    def _():
        o_ref[...] = new


def kernel(x, ln1_w, ln2_w, Wq, Wk, Wv, Wo, bo, Wg, W1, b1, W2, b2):
    x2 = x.reshape(T, N_EMBD)
    ln1 = ln1_w.reshape(1, N_EMBD)
    ln2 = ln2_w.reshape(1, N_EMBD)
    bo2 = bo.reshape(1, N_EMBD)
    wqkv = jnp.concatenate([
        jnp.transpose(Wq, (1, 0, 2)).reshape(N_EMBD, N_HEAD * HEAD_SIZE),
        jnp.transpose(Wk, (1, 0, 2)).reshape(N_EMBD, N_HEAD * HEAD_SIZE),
        jnp.transpose(Wv, (1, 0, 2)).reshape(N_EMBD, N_HEAD * HEAD_SIZE),
    ], axis=1)
    wg_pad = jnp.pad(Wg, ((0, 0), (0, 128 - N_EXPERTS)))

    qkv = pl.pallas_call(
        _qkv_kernel,
        grid=(T // QT,),
        in_specs=[
            pl.BlockSpec((QT, N_EMBD), lambda t: (t, 0)),
            pl.BlockSpec((1, N_EMBD), lambda t: (0, 0)),
            pl.BlockSpec((N_EMBD, 3 * N_EMBD), lambda t: (0, 0)),
        ],
        out_specs=pl.BlockSpec((QT, 3 * N_EMBD), lambda t: (t, 0)),
        out_shape=jax.ShapeDtypeStruct((T, 3 * N_EMBD), jnp.float32),
    )(x2, ln1, wqkv)

    x1, mask, probs8 = pl.pallas_call(
        _attn_gate_kernel,
        grid=(NT_Q,),
        in_specs=[
            pl.BlockSpec((QT, N_EMBD), lambda t: (t, 0)),
            pl.BlockSpec((T, N_EMBD), lambda t: (0, 1)),
            pl.BlockSpec((T, N_EMBD), lambda t: (0, 2)),
            pl.BlockSpec((QT, N_EMBD), lambda t: (t, 0)),
            pl.BlockSpec((N_EMBD, N_EMBD), lambda t: (0, 0)),
            pl.BlockSpec((1, N_EMBD), lambda t: (0, 0)),
            pl.BlockSpec((1, N_EMBD), lambda t: (0, 0)),
            pl.BlockSpec((N_EMBD, 128), lambda t: (0, 0)),
        ],
        out_specs=[
            pl.BlockSpec((QT, N_EMBD), lambda t: (t, 0)),
            pl.BlockSpec((QT, 128), lambda t: (t, 0)),
            pl.BlockSpec((QT, 8), lambda t: (t, 0)),
        ],
        out_shape=[
            jax.ShapeDtypeStruct((T, N_EMBD), jnp.float32),
            jax.ShapeDtypeStruct((T, 128), jnp.float32),
            jax.ShapeDtypeStruct((T, 8), jnp.float32),
        ],
    )(qkv, qkv, qkv, x2, Wo, bo2, ln2, wg_pad)

    probs_tr = jnp.transpose(probs8)[0:3].reshape(3 * T)
    aux16 = _sc_aux(probs_tr)

    out = pl.pallas_call(
        _moe_kernel,
        grid=(N_EXPERTS, NF, NT_M),
        in_specs=[
            pl.BlockSpec((MT, N_EMBD),
                         lambda e, f, t:
                         (jnp.where((e == 0) & (f == 0), t, 0), 0)),
            pl.BlockSpec((1, N_EMBD), lambda e, f, t: (0, 0)),
            pl.BlockSpec((MT, 128), lambda e, f, t: (t, 0)),
            pl.BlockSpec((1, N_EMBD, FC), lambda e, f, t: (e, 0, f)),
            pl.BlockSpec((1, 1, FC), lambda e, f, t: (e, 0, f)),
            pl.BlockSpec((1, FC, N_EMBD), lambda e, f, t: (e, f, 0)),
            pl.BlockSpec((1, 1, N_EMBD), lambda e, f, t: (e, 0, 0)),
        ],
        out_specs=pl.BlockSpec(
            (MT, N_EMBD),
            lambda e, f, t:
            (jnp.where((e == N_EXPERTS - 1) & (f == NF - 1), t, 0), 0)),
        out_shape=jax.ShapeDtypeStruct((T, N_EMBD), jnp.float32),
        scratch_shapes=[pltpu.VMEM((T, N_EMBD), jnp.float32),
                        pltpu.VMEM((T, N_EMBD), jnp.float32)],
    )(x1, ln2, mask, W1, b1.reshape(N_EXPERTS, 1, F), W2,
      b2.reshape(N_EXPERTS, 1, N_EMBD))

    return (out.reshape(1, T, N_EMBD), aux16[0].reshape(()))


# speedup vs baseline: 1.0382x; 1.0001x over previous
import jax
import jax.numpy as jnp
from jax.experimental import pallas as pl
from jax.experimental.pallas import tpu as pltpu
from jax.experimental.pallas import tpu_sc as plsc
import functools

N_EMBD = 768
N_HEAD = 12
HEAD_SIZE = 64
N_EXPERTS = 3
F = 4 * N_EMBD
T = 2048

QT = 512
MT = 256
FC = 1536
NF = F // FC
NEG = -1e30
NT_Q = T // QT
NT_M = T // MT


def _dot(a, b, dims=None):
    if dims is None:
        return jax.lax.dot(a, b, preferred_element_type=jnp.float32)
    return jax.lax.dot_general(a, b, dims,
                               preferred_element_type=jnp.float32)


def _rms(x, w, eps=1e-6):
    return x * jax.lax.rsqrt(jnp.mean(x * x, axis=-1, keepdims=True) + eps) * w


def _qkv_kernel(x_ref, w_ref, wqkv_ref, o_ref):
    xn = _rms(x_ref[...], w_ref[...])
    o_ref[...] = _dot(xn, wqkv_ref[...])


def _attn_gate_kernel(q_ref, k_ref, v_ref, x_ref, wo_ref, bo_ref,
                      ln2_ref, wg_ref, x1_ref, mask_ref, probs_ref):
    qq = q_ref[...]
    kk = k_ref[...]
    vv = v_ref[...]
    outs = []
    for h in range(N_HEAD):
        sl = slice(h * HEAD_SIZE, (h + 1) * HEAD_SIZE)
        q = qq[:, sl]
        k = kk[:, sl]
        v = vv[:, sl]
        s = _dot(q, k, (((1,), (1,)), ((), ()))) * 0.125
        m = jnp.max(s, axis=-1, keepdims=True)
        p = jnp.exp(s - m)
        l = jnp.sum(p, axis=-1, keepdims=True)
        outs.append(_dot(p, v) / l)
    att = jnp.concatenate(outs, axis=1)
    x1 = x_ref[...] + bo_ref[...] + _dot(att, wo_ref[...])
    x1_ref[...] = x1
    h2 = _rms(x1, ln2_ref[...])
    col = jax.lax.broadcasted_iota(jnp.int32, (1, 128), 1)
    logits = _dot(h2, wg_ref[...]) + jnp.where(col < N_EXPERTS, 0.0, NEG)
    m = jnp.max(logits, axis=-1, keepdims=True)
    e = jnp.exp(logits - m)
    probs = e / jnp.sum(e, axis=-1, keepdims=True)
    p0 = probs[:, 0:1]
    p1 = probs[:, 1:2]
    p2 = probs[:, 2:3]
    ex0 = (p1 > p0) & (p2 > p0)
    ex1 = (p0 >= p1) & (p2 > p1)
    ex2 = (p0 >= p2) & (p1 >= p2)
    pex = jnp.where(ex0, p0, jnp.where(ex1, p1, p2))
    denom = (p0 + p1 + p2) - pex
    m0 = jnp.where(ex0, 0.0, p0 / denom)
    m1 = jnp.where(ex1, 0.0, p1 / denom)
    m2 = jnp.where(ex2, 0.0, p2 / denom)
    mask_ref[...] = (jnp.where(col == 0, m0, 0.0) +
                     jnp.where(col == 1, m1, 0.0) +
                     jnp.where(col == 2, m2, 0.0))
    probs_ref[...] = jnp.transpose(probs[:, 0:8])


def _sc_aux(probs_tr_flat):
    info = plsc.get_sparse_core_info()
    nsub = info.num_subcores
    per = T // nsub
    nch = per // 16
    coef = N_EXPERTS * 0.01 / (T * T)
    mesh = plsc.VectorSubcoreMesh(core_axis_name="c", subcore_axis_name="s")

    @functools.partial(
        pl.kernel, mesh=mesh,
        out_type=jax.ShapeDtypeStruct((16,), jnp.float32),
        scratch_types=[
            pltpu.VMEM((per,), jnp.float32),
            pltpu.VMEM((per,), jnp.float32),
            pltpu.VMEM((per,), jnp.float32),
            pltpu.VMEM((96,), jnp.float32),
            pltpu.VMEM((nsub * 96,), jnp.float32),
            pltpu.VMEM((16,), jnp.float32),
            pltpu.VMEM_SHARED((nsub * 96,), jnp.float32),
        ],
    )
    def body(probs_hbm, out_hbm, p0v, p1v, p2v, six_v, gath_v, out_v,
             shared):
        cid = jax.lax.axis_index("c")
        sid = jax.lax.axis_index("s")
        base = sid * per
        pltpu.sync_copy(probs_hbm.at[pl.ds(base, per)], p0v)
        pltpu.sync_copy(probs_hbm.at[pl.ds(T + base, per)], p1v)
        pltpu.sync_copy(probs_hbm.at[pl.ds(2 * T + base, per)], p2v)
        iota = jax.lax.iota(jnp.int32, 16)
        zero = jnp.zeros((16,), jnp.float32)
        one = jnp.ones((16,), jnp.float32)
        sp0 = sp1 = sp2 = st0 = st1 = st2 = zero
        for i in range(nch):
            sl = pl.ds(16 * i, 16)
            p0 = p0v[sl]
            p1 = p1v[sl]
            p2 = p2v[sl]
            sp0 = sp0 + p0
            sp1 = sp1 + p1
            sp2 = sp2 + p2
            g01 = jnp.where(p0 >= p1, one, zero)
            g02 = jnp.where(p0 >= p2, one, zero)
            g12 = jnp.where(p1 >= p2, one, zero)
            t0 = g01 * g02
            t1 = (one - t0) * g12
            t2 = one - t0 - t1
            st0 = st0 + t0
            st1 = st1 + t1
            st2 = st2 + t2
        for k, v in enumerate((sp0, sp1, sp2, st0, st1, st2)):
            six_v[pl.ds(k * 16, 16)] = v
        pltpu.sync_copy(six_v, shared.at[pl.ds(sid * 96, 96)])
        plsc.subcore_barrier()

        @pl.when(jnp.logical_and(cid == 0, sid == 0))
        def _():
            pltpu.sync_copy(shared, gath_v)
            for k in range(6):
                acc = gath_v[pl.ds(k * 16, 16)]
                for j in range(1, nsub):
                    acc = acc + gath_v[pl.ds(j * 96 + k * 16, 16)]
                six_v[pl.ds(k * 16, 16)] = acc
            sums = []
            for k in range(6):
                v = six_v[pl.ds(k * 16, 16)]
                s = v[0]
                for l in range(1, 16):
                    s = s + v[l]
                sums.append(s)
            aux = (sums[0] * sums[3] + sums[1] * sums[4] +
                   sums[2] * sums[5]) * coef
            out_v[...] = jnp.where(iota == 0, aux, 0.0)
            pltpu.sync_copy(out_v, out_hbm)

    return body(probs_tr_flat)


def _moe_kernel(x_ref, w_ref, mask_ref, w1_ref, b1_ref, w2_ref, b2_ref,
                o_ref, acc_ref, h2s_ref):
    e = pl.program_id(0)
    fc = pl.program_id(1)
    t = pl.program_id(2)
    first = jnp.logical_and(e == 0, fc == 0)
    last = jnp.logical_and(e == N_EXPERTS - 1, fc == NF - 1)
    rows = pl.ds(t * MT, MT)
    x = x_ref[...]
    h2_new = _rms(x, w_ref[...])
    h2 = jnp.where(first, h2_new, h2s_ref[rows, :])
    hid = jnp.maximum(_dot(h2, w1_ref[0]) + b1_ref[0], 0.0)
    part = _dot(hid, w2_ref[0])
    part = part + jnp.where(fc == 0, 1.0, 0.0) * b2_ref[0]
    col = jax.lax.broadcasted_iota(jnp.int32, (1, 128), 1)
    msel = jnp.sum(mask_ref[...] * (col == e).astype(jnp.float32),
                   axis=-1, keepdims=True)
    contrib = msel * part
    prev = jnp.where(first, x, acc_ref[rows, :])
    new = prev + contrib
    acc_ref[rows, :] = new

    @pl.when(first)
    def _():
        h2s_ref[rows, :] = h2_new

    @pl.when(last)
    def _():
        o_ref[...] = new


def kernel(x, ln1_w, ln2_w, Wq, Wk, Wv, Wo, bo, Wg, W1, b1, W2, b2):
    x2 = x.reshape(T, N_EMBD)
    ln1 = ln1_w.reshape(1, N_EMBD)
    ln2 = ln2_w.reshape(1, N_EMBD)
    bo2 = bo.reshape(1, N_EMBD)
    wqkv = jnp.concatenate([
        jnp.transpose(Wq, (1, 0, 2)).reshape(N_EMBD, N_HEAD * HEAD_SIZE),
        jnp.transpose(Wk, (1, 0, 2)).reshape(N_EMBD, N_HEAD * HEAD_SIZE),
        jnp.transpose(Wv, (1, 0, 2)).reshape(N_EMBD, N_HEAD * HEAD_SIZE),
    ], axis=1)
    wg_pad = jnp.pad(Wg, ((0, 0), (0, 128 - N_EXPERTS)))

    qkv = pl.pallas_call(
        _qkv_kernel,
        grid=(T // QT,),
        in_specs=[
            pl.BlockSpec((QT, N_EMBD), lambda t: (t, 0)),
            pl.BlockSpec((1, N_EMBD), lambda t: (0, 0)),
            pl.BlockSpec((N_EMBD, 3 * N_EMBD), lambda t: (0, 0)),
        ],
        out_specs=pl.BlockSpec((QT, 3 * N_EMBD), lambda t: (t, 0)),
        out_shape=jax.ShapeDtypeStruct((T, 3 * N_EMBD), jnp.float32),
    )(x2, ln1, wqkv)

    x1, mask, probs8 = pl.pallas_call(
        _attn_gate_kernel,
        grid=(NT_Q,),
        in_specs=[
            pl.BlockSpec((QT, N_EMBD), lambda t: (t, 0)),
            pl.BlockSpec((T, N_EMBD), lambda t: (0, 1)),
            pl.BlockSpec((T, N_EMBD), lambda t: (0, 2)),
            pl.BlockSpec((QT, N_EMBD), lambda t: (t, 0)),
            pl.BlockSpec((N_EMBD, N_EMBD), lambda t: (0, 0)),
            pl.BlockSpec((1, N_EMBD), lambda t: (0, 0)),
            pl.BlockSpec((1, N_EMBD), lambda t: (0, 0)),
            pl.BlockSpec((N_EMBD, 128), lambda t: (0, 0)),
        ],
        out_specs=[
            pl.BlockSpec((QT, N_EMBD), lambda t: (t, 0)),
            pl.BlockSpec((QT, 128), lambda t: (t, 0)),
            pl.BlockSpec((8, QT), lambda t: (0, t)),
        ],
        out_shape=[
            jax.ShapeDtypeStruct((T, N_EMBD), jnp.float32),
            jax.ShapeDtypeStruct((T, 128), jnp.float32),
            jax.ShapeDtypeStruct((8, T), jnp.float32),
        ],
    )(qkv, qkv, qkv, x2, Wo, bo2, ln2, wg_pad)

    aux16 = _sc_aux(probs8.reshape(8 * T))

    out = pl.pallas_call(
        _moe_kernel,
        grid=(N_EXPERTS, NF, NT_M),
        in_specs=[
            pl.BlockSpec((MT, N_EMBD),
                         lambda e, f, t:
                         (jnp.where((e == 0) & (f == 0), t, 0), 0)),
            pl.BlockSpec((1, N_EMBD), lambda e, f, t: (0, 0)),
            pl.BlockSpec((MT, 128), lambda e, f, t: (t, 0)),
            pl.BlockSpec((1, N_EMBD, FC), lambda e, f, t: (e, 0, f)),
            pl.BlockSpec((1, 1, FC), lambda e, f, t: (e, 0, f)),
            pl.BlockSpec((1, FC, N_EMBD), lambda e, f, t: (e, f, 0)),
            pl.BlockSpec((1, 1, N_EMBD), lambda e, f, t: (e, 0, 0)),
        ],
        out_specs=pl.BlockSpec(
            (MT, N_EMBD),
            lambda e, f, t:
            (jnp.where((e == N_EXPERTS - 1) & (f == NF - 1), t, 0), 0)),
        out_shape=jax.ShapeDtypeStruct((T, N_EMBD), jnp.float32),
        scratch_shapes=[pltpu.VMEM((T, N_EMBD), jnp.float32),
                        pltpu.VMEM((T, N_EMBD), jnp.float32)],
    )(x1, ln2, mask, W1, b1.reshape(N_EXPERTS, 1, F), W2,
      b2.reshape(N_EXPERTS, 1, N_EMBD))

    return (out.reshape(1, T, N_EMBD), aux16[0].reshape(()))


# speedup vs baseline: 1.1239x; 1.0825x over previous
import jax
import jax.numpy as jnp
from jax.experimental import pallas as pl
from jax.experimental.pallas import tpu as pltpu

N_EMBD = 768
N_HEAD = 12
HEAD_SIZE = 64
N_EXPERTS = 3
F = 4 * N_EMBD
T = 2048

QT = 256
MT = 256
FC = 1536
NF = F // FC
NEG = -1e30
NT_Q = T // QT
NT_M = T // MT


def _dot(a, b, dims=None):
    if dims is None:
        return jax.lax.dot(a, b, preferred_element_type=jnp.float32)
    return jax.lax.dot_general(a, b, dims,
                               preferred_element_type=jnp.float32)


def _rms(x, w, eps=1e-6):
    return x * jax.lax.rsqrt(jnp.mean(x * x, axis=-1, keepdims=True) + eps) * w


def _attn_gate_kernel(xf_ref, ln1_ref, wqkv_ref, wo_ref, bo_ref,
                      ln2_ref, wg_ref, x1_ref, mask_ref, aux_ref,
                      stat_ref, kv_ref):
    t = pl.program_id(0)

    @pl.when(t == 0)
    def _():
        for c in range(NT_Q):
            crows = pl.ds(c * QT, QT)
            xn_c = _rms(xf_ref[crows, :], ln1_ref[...])
            kv_ref[crows, :] = _dot(xn_c, wqkv_ref[:, N_EMBD:])

    rows = pl.ds(t * QT, QT)
    xt = xf_ref[rows, :]
    qq = _dot(_rms(xt, ln1_ref[...]), wqkv_ref[:, :N_EMBD])
    outs = []
    for h in range(N_HEAD):
        sl = slice(h * HEAD_SIZE, (h + 1) * HEAD_SIZE)
        q = qq[:, sl]
        k = kv_ref[:, sl]
        v = kv_ref[:, N_EMBD + h * HEAD_SIZE:N_EMBD + (h + 1) * HEAD_SIZE]
        s = _dot(q, k, (((1,), (1,)), ((), ()))) * 0.125
        m = jnp.max(s, axis=-1, keepdims=True)
        p = jnp.exp(s - m)
        l = jnp.sum(p, axis=-1, keepdims=True)
        outs.append(_dot(p, v) / l)
    att = jnp.concatenate(outs, axis=1)
    x1 = xt + bo_ref[...] + _dot(att, wo_ref[...])
    x1_ref[...] = x1
    h2 = _rms(x1, ln2_ref[...])
    col = jax.lax.broadcasted_iota(jnp.int32, (1, 128), 1)
    logits = _dot(h2, wg_ref[...]) + jnp.where(col < N_EXPERTS, 0.0, NEG)
    m = jnp.max(logits, axis=-1, keepdims=True)
    e = jnp.exp(logits - m)
    probs = e / jnp.sum(e, axis=-1, keepdims=True)
    p0 = probs[:, 0:1]
    p1 = probs[:, 1:2]
    p2 = probs[:, 2:3]
    ex0 = (p1 > p0) & (p2 > p0)
    ex1 = (p0 >= p1) & (p2 > p1)
    ex2 = (p0 >= p2) & (p1 >= p2)
    pex = jnp.where(ex0, p0, jnp.where(ex1, p1, p2))
    denom = (p0 + p1 + p2) - pex
    m0 = jnp.where(ex0, 0.0, p0 / denom)
    m1 = jnp.where(ex1, 0.0, p1 / denom)
    m2 = jnp.where(ex2, 0.0, p2 / denom)
    mask_ref[...] = (jnp.where(col == 0, m0, 0.0) +
                     jnp.where(col == 1, m1, 0.0) +
                     jnp.where(col == 2, m2, 0.0))
    t0 = (p0 >= p1) & (p0 >= p2)
    t1 = jnp.logical_not(t0) & (p1 >= p2)
    t2 = jnp.logical_not(t0) & jnp.logical_not(t1)
    part = (jnp.where(col == 0, jnp.sum(p0), 0.0) +
            jnp.where(col == 1, jnp.sum(p1), 0.0) +
            jnp.where(col == 2, jnp.sum(p2), 0.0) +
            jnp.where(col == 3, jnp.sum(t0.astype(jnp.float32)), 0.0) +
            jnp.where(col == 4, jnp.sum(t1.astype(jnp.float32)), 0.0) +
            jnp.where(col == 5, jnp.sum(t2.astype(jnp.float32)), 0.0))
    prev = jnp.where(t == 0, jnp.zeros_like(part), stat_ref[...])
    stat = prev + part
    stat_ref[...] = stat

    @pl.when(t == NT_Q - 1)
    def _():
        imp = stat[:, 0:3]
        load = stat[:, 3:6]
        aux = (N_EXPERTS * 0.01 / (T * T)) * jnp.sum(imp * load)
        aux_ref[...] = jnp.full((1, 1), 1.0, jnp.float32) * aux


def _moe_kernel(x_ref, w_ref, mask_ref, w1_ref, b1_ref, w2_ref, b2_ref,
                o_ref, acc_ref, h2s_ref):
    e = pl.program_id(0)
    fc = pl.program_id(1)
    t = pl.program_id(2)
    first = jnp.logical_and(e == 0, fc == 0)
    last = jnp.logical_and(e == N_EXPERTS - 1, fc == NF - 1)
    rows = pl.ds(t * MT, MT)
    x = x_ref[...]
    h2_new = _rms(x, w_ref[...])
    h2 = jnp.where(first, h2_new, h2s_ref[rows, :])
    hid = jnp.maximum(_dot(h2, w1_ref[0]) + b1_ref[0], 0.0)
    part = _dot(hid, w2_ref[0])
    part = part + jnp.where(fc == 0, 1.0, 0.0) * b2_ref[0]
    col = jax.lax.broadcasted_iota(jnp.int32, (1, 128), 1)
    msel = jnp.sum(mask_ref[...] * (col == e).astype(jnp.float32),
                   axis=-1, keepdims=True)
    contrib = msel * part
    prev = jnp.where(first, x, acc_ref[rows, :])
    new = prev + contrib
    acc_ref[rows, :] = new

    @pl.when(first)
    def _():
        h2s_ref[rows, :] = h2_new

    @pl.when(last)
    def _():
        o_ref[...] = new


def kernel(x, ln1_w, ln2_w, Wq, Wk, Wv, Wo, bo, Wg, W1, b1, W2, b2):
    x2 = x.reshape(T, N_EMBD)
    ln1 = ln1_w.reshape(1, N_EMBD)
    ln2 = ln2_w.reshape(1, N_EMBD)
    bo2 = bo.reshape(1, N_EMBD)
    wqkv = jnp.concatenate([
        jnp.transpose(Wq, (1, 0, 2)).reshape(N_EMBD, N_HEAD * HEAD_SIZE),
        jnp.transpose(Wk, (1, 0, 2)).reshape(N_EMBD, N_HEAD * HEAD_SIZE),
        jnp.transpose(Wv, (1, 0, 2)).reshape(N_EMBD, N_HEAD * HEAD_SIZE),
    ], axis=1)
    wg_pad = jnp.pad(Wg, ((0, 0), (0, 128 - N_EXPERTS)))

    x1, mask, aux = pl.pallas_call(
        _attn_gate_kernel,
        grid=(NT_Q,),
        in_specs=[
            pl.BlockSpec((T, N_EMBD), lambda t: (0, 0)),
            pl.BlockSpec((1, N_EMBD), lambda t: (0, 0)),
            pl.BlockSpec((N_EMBD, 3 * N_EMBD), lambda t: (0, 0)),
            pl.BlockSpec((N_EMBD, N_EMBD), lambda t: (0, 0)),
            pl.BlockSpec((1, N_EMBD), lambda t: (0, 0)),
            pl.BlockSpec((1, N_EMBD), lambda t: (0, 0)),
            pl.BlockSpec((N_EMBD, 128), lambda t: (0, 0)),
        ],
        out_specs=[
            pl.BlockSpec((QT, N_EMBD), lambda t: (t, 0)),
            pl.BlockSpec((QT, 128), lambda t: (t, 0)),
            pl.BlockSpec((1, 1), lambda t: (0, 0)),
        ],
        out_shape=[
            jax.ShapeDtypeStruct((T, N_EMBD), jnp.float32),
            jax.ShapeDtypeStruct((T, 128), jnp.float32),
            jax.ShapeDtypeStruct((1, 1), jnp.float32),
        ],
        scratch_shapes=[pltpu.VMEM((1, 128), jnp.float32),
                        pltpu.VMEM((T, 2 * N_EMBD), jnp.float32)],
    )(x2, ln1, wqkv, Wo, bo2, ln2, wg_pad)

    out = pl.pallas_call(
        _moe_kernel,
        grid=(N_EXPERTS, NF, NT_M),
        in_specs=[
            pl.BlockSpec((MT, N_EMBD),
                         lambda e, f, t:
                         (jnp.where((e == 0) & (f == 0), t, 0), 0)),
            pl.BlockSpec((1, N_EMBD), lambda e, f, t: (0, 0)),
            pl.BlockSpec((MT, 128), lambda e, f, t: (t, 0)),
            pl.BlockSpec((1, N_EMBD, FC), lambda e, f, t: (e, 0, f)),
            pl.BlockSpec((1, 1, FC), lambda e, f, t: (e, 0, f)),
            pl.BlockSpec((1, FC, N_EMBD), lambda e, f, t: (e, f, 0)),
            pl.BlockSpec((1, 1, N_EMBD), lambda e, f, t: (e, 0, 0)),
        ],
        out_specs=pl.BlockSpec(
            (MT, N_EMBD),
            lambda e, f, t:
            (jnp.where((e == N_EXPERTS - 1) & (f == NF - 1), t, 0), 0)),
        out_shape=jax.ShapeDtypeStruct((T, N_EMBD), jnp.float32),
        scratch_shapes=[pltpu.VMEM((T, N_EMBD), jnp.float32),
                        pltpu.VMEM((T, N_EMBD), jnp.float32)],
    )(x1, ln2, mask, W1, b1.reshape(N_EXPERTS, 1, F), W2,
      b2.reshape(N_EXPERTS, 1, N_EMBD))

    return (out.reshape(1, T, N_EMBD), aux.reshape(()))


# speedup vs baseline: 1.2188x; 1.0845x over previous
import jax
import jax.numpy as jnp
from jax.experimental import pallas as pl
from jax.experimental.pallas import tpu as pltpu

N_EMBD = 768
N_HEAD = 12
HEAD_SIZE = 64
N_EXPERTS = 3
F = 4 * N_EMBD
T = 2048

QT = 256
MT = 512
FC = 1536
NF = F // FC
NEG = -1e30
NT_Q = T // QT
NT_M = T // MT


def _dot(a, b, dims=None):
    if dims is None:
        return jax.lax.dot(a, b, preferred_element_type=jnp.float32)
    return jax.lax.dot_general(a, b, dims,
                               preferred_element_type=jnp.float32)


def _rms(x, w, eps=1e-6):
    return x * jax.lax.rsqrt(jnp.mean(x * x, axis=-1, keepdims=True) + eps) * w


def _attn_gate_kernel(xf_ref, ln1_ref, wqkv_ref, wo_ref, bo_ref,
                      ln2_ref, wg_ref, x1_ref, mask_ref, aux_ref,
                      stat_ref, kv_ref):
    t = pl.program_id(0)

    @pl.when(t == 0)
    def _():
        for c in range(NT_Q):
            crows = pl.ds(c * QT, QT)
            xn_c = _rms(xf_ref[crows, :], ln1_ref[...])
            kv_ref[crows, :] = _dot(xn_c, wqkv_ref[:, N_EMBD:])

    rows = pl.ds(t * QT, QT)
    xt = xf_ref[rows, :]
    qq = _dot(_rms(xt, ln1_ref[...]), wqkv_ref[:, :N_EMBD])
    outs = []
    for h in range(N_HEAD):
        sl = slice(h * HEAD_SIZE, (h + 1) * HEAD_SIZE)
        q = qq[:, sl]
        k = kv_ref[:, sl]
        v = kv_ref[:, N_EMBD + h * HEAD_SIZE:N_EMBD + (h + 1) * HEAD_SIZE]
        s = _dot(q, k, (((1,), (1,)), ((), ()))) * 0.125
        m = jnp.max(s, axis=-1, keepdims=True)
        p = jnp.exp(s - m)
        l = jnp.sum(p, axis=-1, keepdims=True)
        outs.append(_dot(p, v) / l)
    att = jnp.concatenate(outs, axis=1)
    x1 = xt + bo_ref[...] + _dot(att, wo_ref[...])
    x1_ref[...] = x1
    h2 = _rms(x1, ln2_ref[...])
    col = jax.lax.broadcasted_iota(jnp.int32, (1, 128), 1)
    logits = _dot(h2, wg_ref[...]) + jnp.where(col < N_EXPERTS, 0.0, NEG)
    m = jnp.max(logits, axis=-1, keepdims=True)
    e = jnp.exp(logits - m)
    probs = e / jnp.sum(e, axis=-1, keepdims=True)
    p0 = probs[:, 0:1]
    p1 = probs[:, 1:2]
    p2 = probs[:, 2:3]
    ex0 = (p1 > p0) & (p2 > p0)
    ex1 = (p0 >= p1) & (p2 > p1)
    ex2 = (p0 >= p2) & (p1 >= p2)
    pex = jnp.where(ex0, p0, jnp.where(ex1, p1, p2))
    denom = (p0 + p1 + p2) - pex
    m0 = jnp.where(ex0, 0.0, p0 / denom)
    m1 = jnp.where(ex1, 0.0, p1 / denom)
    m2 = jnp.where(ex2, 0.0, p2 / denom)
    mask_ref[...] = (jnp.where(col == 0, m0, 0.0) +
                     jnp.where(col == 1, m1, 0.0) +
                     jnp.where(col == 2, m2, 0.0))
    t0 = (p0 >= p1) & (p0 >= p2)
    t1 = jnp.logical_not(t0) & (p1 >= p2)
    t2 = jnp.logical_not(t0) & jnp.logical_not(t1)
    part = (jnp.where(col == 0, jnp.sum(p0), 0.0) +
            jnp.where(col == 1, jnp.sum(p1), 0.0) +
            jnp.where(col == 2, jnp.sum(p2), 0.0) +
            jnp.where(col == 3, jnp.sum(t0.astype(jnp.float32)), 0.0) +
            jnp.where(col == 4, jnp.sum(t1.astype(jnp.float32)), 0.0) +
            jnp.where(col == 5, jnp.sum(t2.astype(jnp.float32)), 0.0))
    prev = jnp.where(t == 0, jnp.zeros_like(part), stat_ref[...])
    stat = prev + part
    stat_ref[...] = stat

    @pl.when(t == NT_Q - 1)
    def _():
        imp = stat[:, 0:3]
        load = stat[:, 3:6]
        aux = (N_EXPERTS * 0.01 / (T * T)) * jnp.sum(imp * load)
        aux_ref[...] = jnp.full((1, 1), 1.0, jnp.float32) * aux


def _moe_kernel(x_ref, w_ref, mask_ref, w1_ref, b1_ref, w2_ref, b2_ref,
                o_ref, acc_ref, h2s_ref):
    e = pl.program_id(0)
    fc = pl.program_id(1)
    t = pl.program_id(2)
    first = jnp.logical_and(e == 0, fc == 0)
    last = jnp.logical_and(e == N_EXPERTS - 1, fc == NF - 1)
    rows = pl.ds(t * MT, MT)
    x = x_ref[...]
    h2_new = _rms(x, w_ref[...])
    h2 = jnp.where(first, h2_new, h2s_ref[rows, :])
    hid = jnp.maximum(_dot(h2, w1_ref[0]) + b1_ref[0], 0.0)
    part = _dot(hid, w2_ref[0])
    part = part + jnp.where(fc == 0, 1.0, 0.0) * b2_ref[0]
    col = jax.lax.broadcasted_iota(jnp.int32, (1, 128), 1)
    msel = jnp.sum(mask_ref[...] * (col == e).astype(jnp.float32),
                   axis=-1, keepdims=True)
    contrib = msel * part
    prev = jnp.where(first, x, acc_ref[rows, :])
    new = prev + contrib
    acc_ref[rows, :] = new

    @pl.when(first)
    def _():
        h2s_ref[rows, :] = h2_new

    @pl.when(last)
    def _():
        o_ref[...] = new


def kernel(x, ln1_w, ln2_w, Wq, Wk, Wv, Wo, bo, Wg, W1, b1, W2, b2):
    x2 = x.reshape(T, N_EMBD)
    ln1 = ln1_w.reshape(1, N_EMBD)
    ln2 = ln2_w.reshape(1, N_EMBD)
    bo2 = bo.reshape(1, N_EMBD)
    wqkv = jnp.concatenate([
        jnp.transpose(Wq, (1, 0, 2)).reshape(N_EMBD, N_HEAD * HEAD_SIZE),
        jnp.transpose(Wk, (1, 0, 2)).reshape(N_EMBD, N_HEAD * HEAD_SIZE),
        jnp.transpose(Wv, (1, 0, 2)).reshape(N_EMBD, N_HEAD * HEAD_SIZE),
    ], axis=1)
    wg_pad = jnp.pad(Wg, ((0, 0), (0, 128 - N_EXPERTS)))

    x1, mask, aux = pl.pallas_call(
        _attn_gate_kernel,
        grid=(NT_Q,),
        in_specs=[
            pl.BlockSpec((T, N_EMBD), lambda t: (0, 0)),
            pl.BlockSpec((1, N_EMBD), lambda t: (0, 0)),
            pl.BlockSpec((N_EMBD, 3 * N_EMBD), lambda t: (0, 0)),
            pl.BlockSpec((N_EMBD, N_EMBD), lambda t: (0, 0)),
            pl.BlockSpec((1, N_EMBD), lambda t: (0, 0)),
            pl.BlockSpec((1, N_EMBD), lambda t: (0, 0)),
            pl.BlockSpec((N_EMBD, 128), lambda t: (0, 0)),
        ],
        out_specs=[
            pl.BlockSpec((QT, N_EMBD), lambda t: (t, 0)),
            pl.BlockSpec((QT, 128), lambda t: (t, 0)),
            pl.BlockSpec((1, 1), lambda t: (0, 0)),
        ],
        out_shape=[
            jax.ShapeDtypeStruct((T, N_EMBD), jnp.float32),
            jax.ShapeDtypeStruct((T, 128), jnp.float32),
            jax.ShapeDtypeStruct((1, 1), jnp.float32),
        ],
        scratch_shapes=[pltpu.VMEM((1, 128), jnp.float32),
                        pltpu.VMEM((T, 2 * N_EMBD), jnp.float32)],
    )(x2, ln1, wqkv, Wo, bo2, ln2, wg_pad)

    out = pl.pallas_call(
        _moe_kernel,
        grid=(N_EXPERTS, NF, NT_M),
        in_specs=[
            pl.BlockSpec((MT, N_EMBD),
                         lambda e, f, t:
                         (jnp.where((e == 0) & (f == 0), t, 0), 0)),
            pl.BlockSpec((1, N_EMBD), lambda e, f, t: (0, 0)),
            pl.BlockSpec((MT, 128), lambda e, f, t: (t, 0)),
            pl.BlockSpec((1, N_EMBD, FC), lambda e, f, t: (e, 0, f)),
            pl.BlockSpec((1, 1, FC), lambda e, f, t: (e, 0, f)),
            pl.BlockSpec((1, FC, N_EMBD), lambda e, f, t: (e, f, 0)),
            pl.BlockSpec((1, 1, N_EMBD), lambda e, f, t: (e, 0, 0)),
        ],
        out_specs=pl.BlockSpec(
            (MT, N_EMBD),
            lambda e, f, t:
            (jnp.where((e == N_EXPERTS - 1) & (f == NF - 1), t, 0), 0)),
        out_shape=jax.ShapeDtypeStruct((T, N_EMBD), jnp.float32),
        scratch_shapes=[pltpu.VMEM((T, N_EMBD), jnp.float32),
                        pltpu.VMEM((T, N_EMBD), jnp.float32)],
    )(x1, ln2, mask, W1, b1.reshape(N_EXPERTS, 1, F), W2,
      b2.reshape(N_EXPERTS, 1, N_EMBD))

    return (out.reshape(1, T, N_EMBD), aux.reshape(()))


# speedup vs baseline: 1.2733x; 1.0447x over previous
import jax
import jax.numpy as jnp
from jax.experimental import pallas as pl
from jax.experimental.pallas import tpu as pltpu

N_EMBD = 768
N_HEAD = 12
HEAD_SIZE = 64
N_EXPERTS = 3
F = 4 * N_EMBD
T = 2048

QT = 256
MT = 1024
FC = 1536
NF = F // FC
NEG = -1e30
NT_Q = T // QT
NT_M = T // MT


def _dot(a, b, dims=None):
    if dims is None:
        return jax.lax.dot(a, b, preferred_element_type=jnp.float32)
    return jax.lax.dot_general(a, b, dims,
                               preferred_element_type=jnp.float32)


def _rms(x, w, eps=1e-6):
    return x * jax.lax.rsqrt(jnp.mean(x * x, axis=-1, keepdims=True) + eps) * w


def _attn_gate_kernel(xf_ref, ln1_ref, wqkv_ref, wo_ref, bo_ref,
                      ln2_ref, wg_ref, x1_ref, mask_ref, aux_ref,
                      stat_ref, kv_ref):
    t = pl.program_id(0)

    @pl.when(t == 0)
    def _():
        for c in range(NT_Q):
            crows = pl.ds(c * QT, QT)
            xn_c = _rms(xf_ref[crows, :], ln1_ref[...])
            kv_ref[crows, :] = _dot(xn_c, wqkv_ref[:, N_EMBD:])

    rows = pl.ds(t * QT, QT)
    xt = xf_ref[rows, :]
    qq = _dot(_rms(xt, ln1_ref[...]), wqkv_ref[:, :N_EMBD])
    outs = []
    for h in range(N_HEAD):
        sl = slice(h * HEAD_SIZE, (h + 1) * HEAD_SIZE)
        q = qq[:, sl]
        k = kv_ref[:, sl]
        v = kv_ref[:, N_EMBD + h * HEAD_SIZE:N_EMBD + (h + 1) * HEAD_SIZE]
        s = _dot(q, k, (((1,), (1,)), ((), ()))) * 0.125
        m = jnp.max(s, axis=-1, keepdims=True)
        p = jnp.exp(s - m)
        l = jnp.sum(p, axis=-1, keepdims=True)
        outs.append(_dot(p, v) / l)
    att = jnp.concatenate(outs, axis=1)
    x1 = xt + bo_ref[...] + _dot(att, wo_ref[...])
    x1_ref[...] = x1
    h2 = _rms(x1, ln2_ref[...])
    col = jax.lax.broadcasted_iota(jnp.int32, (1, 128), 1)
    logits = _dot(h2, wg_ref[...]) + jnp.where(col < N_EXPERTS, 0.0, NEG)
    m = jnp.max(logits, axis=-1, keepdims=True)
    e = jnp.exp(logits - m)
    probs = e / jnp.sum(e, axis=-1, keepdims=True)
    p0 = probs[:, 0:1]
    p1 = probs[:, 1:2]
    p2 = probs[:, 2:3]
    ex0 = (p1 > p0) & (p2 > p0)
    ex1 = (p0 >= p1) & (p2 > p1)
    ex2 = (p0 >= p2) & (p1 >= p2)
    pex = jnp.where(ex0, p0, jnp.where(ex1, p1, p2))
    denom = (p0 + p1 + p2) - pex
    m0 = jnp.where(ex0, 0.0, p0 / denom)
    m1 = jnp.where(ex1, 0.0, p1 / denom)
    m2 = jnp.where(ex2, 0.0, p2 / denom)
    mask_ref[...] = (jnp.where(col == 0, m0, 0.0) +
                     jnp.where(col == 1, m1, 0.0) +
                     jnp.where(col == 2, m2, 0.0))
    t0 = (p0 >= p1) & (p0 >= p2)
    t1 = jnp.logical_not(t0) & (p1 >= p2)
    t2 = jnp.logical_not(t0) & jnp.logical_not(t1)
    part = (jnp.where(col == 0, jnp.sum(p0), 0.0) +
            jnp.where(col == 1, jnp.sum(p1), 0.0) +
            jnp.where(col == 2, jnp.sum(p2), 0.0) +
            jnp.where(col == 3, jnp.sum(t0.astype(jnp.float32)), 0.0) +
            jnp.where(col == 4, jnp.sum(t1.astype(jnp.float32)), 0.0) +
            jnp.where(col == 5, jnp.sum(t2.astype(jnp.float32)), 0.0))
    prev = jnp.where(t == 0, jnp.zeros_like(part), stat_ref[...])
    stat = prev + part
    stat_ref[...] = stat

    @pl.when(t == NT_Q - 1)
    def _():
        imp = stat[:, 0:3]
        load = stat[:, 3:6]
        aux = (N_EXPERTS * 0.01 / (T * T)) * jnp.sum(imp * load)
        aux_ref[...] = jnp.full((1, 1), 1.0, jnp.float32) * aux


def _moe_kernel(x_ref, w_ref, mask_ref, w1_ref, b1_ref, w2_ref, b2_ref,
                o_ref, acc_ref, h2s_ref):
    e = pl.program_id(0)
    fc = pl.program_id(1)
    t = pl.program_id(2)
    first = jnp.logical_and(e == 0, fc == 0)
    last = jnp.logical_and(e == N_EXPERTS - 1, fc == NF - 1)
    rows = pl.ds(t * MT, MT)
    x = x_ref[...]
    h2_new = _rms(x, w_ref[...])
    h2 = jnp.where(first, h2_new, h2s_ref[rows, :])
    hid = jnp.maximum(_dot(h2, w1_ref[0]) + b1_ref[0], 0.0)
    part = _dot(hid, w2_ref[0])
    part = part + jnp.where(fc == 0, 1.0, 0.0) * b2_ref[0]
    col = jax.lax.broadcasted_iota(jnp.int32, (1, 128), 1)
    msel = jnp.sum(mask_ref[...] * (col == e).astype(jnp.float32),
                   axis=-1, keepdims=True)
    contrib = msel * part
    prev = jnp.where(first, x, acc_ref[rows, :])
    new = prev + contrib
    acc_ref[rows, :] = new

    @pl.when(first)
    def _():
        h2s_ref[rows, :] = h2_new

    @pl.when(last)
    def _():
        o_ref[...] = new


def kernel(x, ln1_w, ln2_w, Wq, Wk, Wv, Wo, bo, Wg, W1, b1, W2, b2):
    x2 = x.reshape(T, N_EMBD)
    ln1 = ln1_w.reshape(1, N_EMBD)
    ln2 = ln2_w.reshape(1, N_EMBD)
    bo2 = bo.reshape(1, N_EMBD)
    wqkv = jnp.concatenate([
        jnp.transpose(Wq, (1, 0, 2)).reshape(N_EMBD, N_HEAD * HEAD_SIZE),
        jnp.transpose(Wk, (1, 0, 2)).reshape(N_EMBD, N_HEAD * HEAD_SIZE),
        jnp.transpose(Wv, (1, 0, 2)).reshape(N_EMBD, N_HEAD * HEAD_SIZE),
    ], axis=1)
    wg_pad = jnp.pad(Wg, ((0, 0), (0, 128 - N_EXPERTS)))

    x1, mask, aux = pl.pallas_call(
        _attn_gate_kernel,
        grid=(NT_Q,),
        in_specs=[
            pl.BlockSpec((T, N_EMBD), lambda t: (0, 0)),
            pl.BlockSpec((1, N_EMBD), lambda t: (0, 0)),
            pl.BlockSpec((N_EMBD, 3 * N_EMBD), lambda t: (0, 0)),
            pl.BlockSpec((N_EMBD, N_EMBD), lambda t: (0, 0)),
            pl.BlockSpec((1, N_EMBD), lambda t: (0, 0)),
            pl.BlockSpec((1, N_EMBD), lambda t: (0, 0)),
            pl.BlockSpec((N_EMBD, 128), lambda t: (0, 0)),
        ],
        out_specs=[
            pl.BlockSpec((QT, N_EMBD), lambda t: (t, 0)),
            pl.BlockSpec((QT, 128), lambda t: (t, 0)),
            pl.BlockSpec((1, 1), lambda t: (0, 0)),
        ],
        out_shape=[
            jax.ShapeDtypeStruct((T, N_EMBD), jnp.float32),
            jax.ShapeDtypeStruct((T, 128), jnp.float32),
            jax.ShapeDtypeStruct((1, 1), jnp.float32),
        ],
        scratch_shapes=[pltpu.VMEM((1, 128), jnp.float32),
                        pltpu.VMEM((T, 2 * N_EMBD), jnp.float32)],
    )(x2, ln1, wqkv, Wo, bo2, ln2, wg_pad)

    out = pl.pallas_call(
        _moe_kernel,
        grid=(N_EXPERTS, NF, NT_M),
        in_specs=[
            pl.BlockSpec((MT, N_EMBD),
                         lambda e, f, t:
                         (jnp.where((e == 0) & (f == 0), t, 0), 0)),
            pl.BlockSpec((1, N_EMBD), lambda e, f, t: (0, 0)),
            pl.BlockSpec((MT, 128), lambda e, f, t: (t, 0)),
            pl.BlockSpec((1, N_EMBD, FC), lambda e, f, t: (e, 0, f)),
            pl.BlockSpec((1, 1, FC), lambda e, f, t: (e, 0, f)),
            pl.BlockSpec((1, FC, N_EMBD), lambda e, f, t: (e, f, 0)),
            pl.BlockSpec((1, 1, N_EMBD), lambda e, f, t: (e, 0, 0)),
        ],
        out_specs=pl.BlockSpec(
            (MT, N_EMBD),
            lambda e, f, t:
            (jnp.where((e == N_EXPERTS - 1) & (f == NF - 1), t, 0), 0)),
        out_shape=jax.ShapeDtypeStruct((T, N_EMBD), jnp.float32),
        scratch_shapes=[pltpu.VMEM((T, N_EMBD), jnp.float32),
                        pltpu.VMEM((T, N_EMBD), jnp.float32)],
    )(x1, ln2, mask, W1, b1.reshape(N_EXPERTS, 1, F), W2,
      b2.reshape(N_EXPERTS, 1, N_EMBD))

    return (out.reshape(1, T, N_EMBD), aux.reshape(()))
